# Initial kernel scaffold; baseline (speedup 1.0000x reference)
import jax
import jax.numpy as jnp
from jax.experimental import pallas as pl


def kernel(beam_tokens, beam_scores, token_weights, beam_prev_indices, num_steps):
    raise NotImplementedError("write your pallas kernel here")


# speedup vs baseline: 156.0057x; 156.0057x over previous
import functools

import jax
import jax.numpy as jnp
from jax import lax
from jax.experimental import pallas as pl
from jax.experimental.pallas import tpu as pltpu
from jax.experimental.pallas import tpu_sc as plsc

T = 2048
BEAM = 8
SRC = 2048
NBEST = 4
NS = T - 1
ROWS = NBEST * T

_MESH = dict(core_axis_name="c", subcore_axis_name="s", num_cores=2,
             num_subcores=16)


def _stage1_body(tokens_hbm, scores_hbm, prev_hbm,
                 ord_hbm, sc_hbm, tok_hbm, tls_hbm, rows_hbm,
                 tokens_v, scores_v, prev_v,
                 rows_v, tokbuf_v, scorebuf_v, tlsbuf_v, misci_v, miscf_v):
    cid = lax.axis_index("c")
    sid = lax.axis_index("s")

    @pl.when(jnp.logical_and(cid == 0, sid == 0))
    def _():
        pltpu.sync_copy(tokens_hbm, tokens_v)
        pltpu.sync_copy(scores_hbm, scores_v)
        pltpu.sync_copy(prev_hbm, prev_v)

        lane = lax.broadcasted_iota(jnp.int32, (16,), 0)
        mask4 = lane < NBEST

        sc_last = scores_v[pl.ds(T * BEAM - 16, 16)]
        neg = jnp.float32(-jnp.inf)
        cand = jnp.where(lane >= 8, sc_last, neg)
        b = jnp.zeros((16,), jnp.int32)
        for i in range(NBEST):
            m = cand
            for sh in (1, 2, 4, 8):
                rot = m.at[jnp.bitwise_and(lane + sh, 15)].get(
                    mode="promise_in_bounds")
                m = jnp.maximum(m, rot)
            j = plsc.all_reduce_ffs(cand == m)
            b = jnp.where(lane == i, j - 8, b)
            cand = jnp.where(lane == j, neg, cand)

        misci_v[...] = jnp.where(mask4, b, 0)
        sc4 = plsc.load_gather(scores_v, [NS * BEAM + b])
        miscf_v[...] = jnp.where(mask4, sc4, jnp.float32(0.0))
        pltpu.sync_copy(misci_v, ord_hbm)
        pltpu.sync_copy(miscf_v, sc_hbm)

        base4 = lane * T
        zi = jnp.zeros((16,), jnp.int32)
        plsc.store_scatter(scorebuf_v, [base4], jnp.zeros((16,), jnp.float32),
                           mask=mask4)
        plsc.store_scatter(rows_v, [base4 + NS], zi, mask=mask4)
        plsc.store_scatter(tokbuf_v, [base4 + NS], zi, mask=mask4)

        def bt_body(k, bcur):
            t = NS - k
            idx = t * BEAM + bcur
            plsc.store_scatter(rows_v, [base4 + (t - 1)], idx, mask=mask4)
            tok = plsc.load_gather(tokens_v, [idx])
            plsc.store_scatter(tokbuf_v, [base4 + (t - 1)], tok, mask=mask4)
            sc = plsc.load_gather(scores_v, [idx])
            plsc.store_scatter(scorebuf_v, [base4 + t], sc, mask=mask4)
            return plsc.load_gather(prev_v, [idx])

        lax.fori_loop(0, NS, bt_body, b)

        def tls_body(v, _):
            off = (v // 128) * T + (v % 128) * 16
            a = scorebuf_v[pl.ds(off + 1, 16)]
            c = scorebuf_v[pl.ds(off, 16)]
            tlsbuf_v[pl.ds(off, 16)] = a - c
            return 0

        lax.fori_loop(0, NBEST * 128, tls_body, 0)

        pltpu.sync_copy(tokbuf_v, tok_hbm)
        pltpu.sync_copy(tlsbuf_v, tls_hbm)
        pltpu.sync_copy(rows_v, rows_hbm)


_stage1 = functools.partial(
    pl.kernel,
    out_type=[
        jax.ShapeDtypeStruct((16,), jnp.int32),
        jax.ShapeDtypeStruct((16,), jnp.float32),
        jax.ShapeDtypeStruct((ROWS,), jnp.int32),
        jax.ShapeDtypeStruct((ROWS,), jnp.float32),
        jax.ShapeDtypeStruct((ROWS,), jnp.int32),
    ],
    mesh=plsc.VectorSubcoreMesh(**_MESH),
    compiler_params=pltpu.CompilerParams(needs_layout_passes=False),
    scratch_types=[
        pltpu.VMEM((T * BEAM,), jnp.int32),
        pltpu.VMEM((T * BEAM,), jnp.float32),
        pltpu.VMEM((T * BEAM,), jnp.int32),
        pltpu.VMEM((ROWS,), jnp.int32),
        pltpu.VMEM((ROWS,), jnp.int32),
        pltpu.VMEM((ROWS + 16,), jnp.float32),
        pltpu.VMEM((ROWS,), jnp.float32),
        pltpu.VMEM((16,), jnp.int32),
        pltpu.VMEM((16,), jnp.float32),
    ],
)(_stage1_body)


_CHUNK = 16
_PER_W = ROWS // 32


def _stage2_body(tw_hbm, rows_hbm, out_hbm, idx_v, buf_v, sem):
    wid = lax.axis_index("s") * 2 + lax.axis_index("c")
    base = wid * _PER_W
    pltpu.sync_copy(rows_hbm.at[pl.ds(base, _PER_W)], idx_v)

    def chunk(c, _):
        cp = pltpu.async_copy(
            tw_hbm.at[idx_v.at[pl.ds(c * _CHUNK, _CHUNK)]], buf_v, sem)
        cp.wait()
        pltpu.sync_copy(buf_v, out_hbm.at[pl.ds(base + c * _CHUNK, _CHUNK)])
        return 0

    lax.fori_loop(0, _PER_W // _CHUNK, chunk, 0)


_stage2 = functools.partial(
    pl.kernel,
    out_type=jax.ShapeDtypeStruct((ROWS, SRC), jnp.float32),
    mesh=plsc.VectorSubcoreMesh(**_MESH),
    compiler_params=pltpu.CompilerParams(needs_layout_passes=False),
    scratch_types=[
        pltpu.VMEM((_PER_W,), jnp.int32),
        pltpu.VMEM((_CHUNK, SRC), jnp.float32),
        pltpu.SemaphoreType.DMA,
    ],
)(_stage2_body)


def _tr_body(x_ref, o_ref):
    o_ref[0] = jnp.swapaxes(x_ref[0], 0, 1)


def _stage3(compact):
    return pl.pallas_call(
        _tr_body,
        grid=(NBEST, SRC // 128),
        in_specs=[pl.BlockSpec((1, T, 128), lambda i, s: (i, 0, s))],
        out_specs=pl.BlockSpec((1, 128, T), lambda i, s: (i, s, 0)),
        out_shape=jax.ShapeDtypeStruct((NBEST, SRC, T), jnp.float32),
    )(compact)


def kernel(beam_tokens, beam_scores, token_weights, beam_prev_indices,
           num_steps):
    tokens_flat = beam_tokens.reshape(-1)
    scores_flat = beam_scores.reshape(-1)
    prev_flat = beam_prev_indices.reshape(-1)
    tw_flat = token_weights.reshape(T * BEAM, SRC)

    ord16, sc16, tokf, tlsf, rows = _stage1(tokens_flat, scores_flat,
                                            prev_flat)
    compact = _stage2(tw_flat, rows)
    trans = _stage3(compact.reshape(NBEST, T, SRC))

    ns_t = jnp.asarray(num_steps, jnp.int32)
    ns_f = ns_t.astype(jnp.float32)
    tok4 = tokf.reshape(NBEST, T)
    tls4 = tlsf.reshape(NBEST, T)
    outs = []
    for i in range(NBEST):
        outs.extend([
            tok4[i, :NS],
            sc16[i] / ns_f,
            tls4[i, :NS],
            trans[i, :, :NS],
            jnp.stack([ns_t, ord16[i]]).astype(jnp.int32),
        ])
    return tuple(outs)


# speedup vs baseline: 245.0629x; 1.5709x over previous
import functools

import jax
import jax.numpy as jnp
from jax import lax
from jax.experimental import pallas as pl
from jax.experimental.pallas import tpu as pltpu
from jax.experimental.pallas import tpu_sc as plsc

T = 2048
BEAM = 8
SRC = 2048
NBEST = 4
NS = T - 1
ROWS = NBEST * T

_MESH = dict(core_axis_name="c", subcore_axis_name="s", num_cores=2,
             num_subcores=16)


def _stage1_body(tokens_hbm, scores_hbm, prev_hbm,
                 ord_hbm, sc_hbm, tok_hbm, tls_hbm, rows_hbm,
                 tokens_v, scores_v, prev_v,
                 rows_v, tokbuf_v, tlsbuf_v, misci_v, miscf_v):
    cid = lax.axis_index("c")
    sid = lax.axis_index("s")

    @pl.when(jnp.logical_and(cid == 0, sid == 0))
    def _():
        pltpu.sync_copy(tokens_hbm, tokens_v)
        pltpu.sync_copy(scores_hbm, scores_v)
        pltpu.sync_copy(prev_hbm, prev_v)

        lane = lax.broadcasted_iota(jnp.int32, (16,), 0)
        mask4 = lane < NBEST

        sc_last = scores_v[pl.ds(T * BEAM - 16, 16)]
        neg = jnp.float32(-jnp.inf)
        cand = jnp.where(lane >= 8, sc_last, neg)
        b = jnp.zeros((16,), jnp.int32)
        for i in range(NBEST):
            m = cand
            for sh in (1, 2, 4, 8):
                rot = m.at[jnp.bitwise_and(lane + sh, 15)].get(
                    mode="promise_in_bounds")
                m = jnp.maximum(m, rot)
            j = plsc.all_reduce_ffs(cand == m)
            b = jnp.where(lane == i, j - 8, b)
            cand = jnp.where(lane == j, neg, cand)

        misci_v[...] = jnp.where(mask4, b, 0)
        sc4 = plsc.load_gather(scores_v, [NS * BEAM + b])
        miscf_v[...] = jnp.where(mask4, sc4, jnp.float32(0.0))
        pltpu.sync_copy(misci_v, ord_hbm)
        pltpu.sync_copy(miscf_v, sc_hbm)

        base4 = lane * T
        plsc.store_scatter(rows_v, [base4 + NS], jnp.zeros((16,), jnp.int32),
                           mask=mask4)

        def bt_body(k, bcur):
            t = NS - k
            idx = t * BEAM + bcur
            plsc.store_scatter(rows_v, [base4 + (t - 1)], idx, mask=mask4)
            return plsc.load_gather(prev_v, [idx])

        lax.fori_loop(0, NS, bt_body, b)

        rotm1 = jnp.bitwise_and(lane + 15, 16 - 1)
        for i in range(NBEST):
            def blk_body(v, carry, i=i):
                off = i * T + v * 16
                ivec = rows_v[pl.ds(off, 16)]
                tokbuf_v[pl.ds(off, 16)] = plsc.load_gather(tokens_v, [ivec])
                sc = plsc.load_gather(scores_v, [ivec])
                srot = sc.at[rotm1].get(mode="promise_in_bounds")
                prev_sc = jnp.where(lane == 0, carry, srot)
                tlsbuf_v[pl.ds(off, 16)] = sc - prev_sc
                return sc[15]

            lax.fori_loop(0, T // 16, blk_body, jnp.float32(0.0))

        pltpu.sync_copy(tokbuf_v, tok_hbm)
        pltpu.sync_copy(tlsbuf_v, tls_hbm)
        pltpu.sync_copy(rows_v, rows_hbm)


_stage1 = functools.partial(
    pl.kernel,
    out_type=[
        jax.ShapeDtypeStruct((16,), jnp.int32),
        jax.ShapeDtypeStruct((16,), jnp.float32),
        jax.ShapeDtypeStruct((ROWS,), jnp.int32),
        jax.ShapeDtypeStruct((ROWS,), jnp.float32),
        jax.ShapeDtypeStruct((ROWS,), jnp.int32),
    ],
    mesh=plsc.VectorSubcoreMesh(**_MESH),
    compiler_params=pltpu.CompilerParams(needs_layout_passes=False),
    scratch_types=[
        pltpu.VMEM((T * BEAM,), jnp.int32),
        pltpu.VMEM((T * BEAM,), jnp.float32),
        pltpu.VMEM((T * BEAM,), jnp.int32),
        pltpu.VMEM((ROWS,), jnp.int32),
        pltpu.VMEM((ROWS,), jnp.int32),
        pltpu.VMEM((ROWS,), jnp.float32),
        pltpu.VMEM((16,), jnp.int32),
        pltpu.VMEM((16,), jnp.float32),
    ],
)(_stage1_body)


_CHUNK = 16
_PER_W = ROWS // 32
_NCHUNK = _PER_W // _CHUNK


def _stage2_body(tw_hbm, rows_hbm, out_hbm, idx_v, buf0, buf1, sem0, sem1):
    wid = lax.axis_index("s") * 2 + lax.axis_index("c")
    base = wid * _PER_W
    pltpu.sync_copy(rows_hbm.at[pl.ds(base, _PER_W)], idx_v)

    bufs = (buf0, buf1)
    sems = (sem0, sem1)

    def fire(c):
        return pltpu.async_copy(
            tw_hbm.at[idx_v.at[pl.ds(c * _CHUNK, _CHUNK)]],
            bufs[c & 1], sems[c & 1])

    cps = {0: fire(0)}
    for c in range(_NCHUNK):
        if c + 1 < _NCHUNK:
            cps[(c + 1) & 1] = fire(c + 1)
        cps[c & 1].wait()
        pltpu.sync_copy(bufs[c & 1],
                        out_hbm.at[pl.ds(base + c * _CHUNK, _CHUNK)])


_stage2 = functools.partial(
    pl.kernel,
    out_type=jax.ShapeDtypeStruct((ROWS, SRC), jnp.float32),
    mesh=plsc.VectorSubcoreMesh(**_MESH),
    compiler_params=pltpu.CompilerParams(needs_layout_passes=False),
    scratch_types=[
        pltpu.VMEM((_PER_W,), jnp.int32),
        pltpu.VMEM((_CHUNK, SRC), jnp.float32),
        pltpu.VMEM((_CHUNK, SRC), jnp.float32),
        pltpu.SemaphoreType.DMA,
        pltpu.SemaphoreType.DMA,
    ],
)(_stage2_body)


def _tr_body(x0, x1, x2, x3, o0, o1, o2, o3):
    for x, o in ((x0, o0), (x1, o1), (x2, o2), (x3, o3)):
        o[...] = jnp.swapaxes(x[0], 0, 1)[:, :NS]


def _stage3(compact):
    in_specs = [
        pl.BlockSpec((1, T, 128), lambda s, k=k: (k, 0, s))
        for k in range(NBEST)
    ]
    out_specs = [
        pl.BlockSpec((128, NS), lambda s: (s, 0)) for _ in range(NBEST)
    ]
    out_shape = [
        jax.ShapeDtypeStruct((SRC, NS), jnp.float32) for _ in range(NBEST)
    ]
    return pl.pallas_call(
        _tr_body,
        grid=(SRC // 128,),
        in_specs=in_specs,
        out_specs=out_specs,
        out_shape=out_shape,
    )(compact, compact, compact, compact)


def kernel(beam_tokens, beam_scores, token_weights, beam_prev_indices,
           num_steps):
    tokens_flat = beam_tokens.reshape(-1)
    scores_flat = beam_scores.reshape(-1)
    prev_flat = beam_prev_indices.reshape(-1)
    tw_flat = token_weights.reshape(T * BEAM, SRC)

    ord16, sc16, tokf, tlsf, rows = _stage1(tokens_flat, scores_flat,
                                            prev_flat)
    compact = _stage2(tw_flat, rows)
    baw = _stage3(compact.reshape(NBEST, T, SRC))

    ns_t = jnp.asarray(num_steps, jnp.int32)
    ns_f = ns_t.astype(jnp.float32)
    tok4 = tokf.reshape(NBEST, T)
    tls4 = tlsf.reshape(NBEST, T)
    outs = []
    for i in range(NBEST):
        outs.extend([
            tok4[i, :NS],
            sc16[i] / ns_f,
            tls4[i, :NS],
            baw[i],
            jnp.stack([ns_t, ord16[i]]).astype(jnp.int32),
        ])
    return tuple(outs)


# speedup vs baseline: 249.4061x; 1.0177x over previous
import functools

import jax
import jax.numpy as jnp
from jax import lax
from jax.experimental import pallas as pl
from jax.experimental.pallas import tpu as pltpu
from jax.experimental.pallas import tpu_sc as plsc

T = 2048
BEAM = 8
SRC = 2048
NBEST = 4
NS = T - 1
ROWS = NBEST * T

_MESH = dict(core_axis_name="c", subcore_axis_name="s", num_cores=2,
             num_subcores=16)


def _stage1_body(tokens_hbm, scores_hbm, prev_hbm,
                 ord_hbm, sc_hbm, tok_hbm, tls_hbm, rows_hbm,
                 tokens_v, scores_v, prev_v,
                 rows_v, tokbuf_v, tlsbuf_v, misci_v, miscf_v):
    cid = lax.axis_index("c")
    sid = lax.axis_index("s")

    @pl.when(jnp.logical_and(cid == 0, sid == 0))
    def _():
        pltpu.sync_copy(tokens_hbm, tokens_v)
        pltpu.sync_copy(scores_hbm, scores_v)
        pltpu.sync_copy(prev_hbm, prev_v)

        lane = lax.broadcasted_iota(jnp.int32, (16,), 0)
        mask4 = lane < NBEST

        sc_last = scores_v[pl.ds(T * BEAM - 16, 16)]
        neg = jnp.float32(-jnp.inf)
        cand = jnp.where(lane >= 8, sc_last, neg)
        b = jnp.zeros((16,), jnp.int32)
        for i in range(NBEST):
            m = cand
            for sh in (1, 2, 4, 8):
                rot = m.at[jnp.bitwise_and(lane + sh, 15)].get(
                    mode="promise_in_bounds")
                m = jnp.maximum(m, rot)
            j = plsc.all_reduce_ffs(cand == m)
            b = jnp.where(lane == i, j - 8, b)
            cand = jnp.where(lane == j, neg, cand)

        misci_v[...] = jnp.where(mask4, b, 0)
        sc4 = plsc.load_gather(scores_v, [NS * BEAM + b])
        miscf_v[...] = jnp.where(mask4, sc4, jnp.float32(0.0))
        pltpu.sync_copy(misci_v, ord_hbm)
        pltpu.sync_copy(miscf_v, sc_hbm)

        base4 = lane * T
        plsc.store_scatter(rows_v, [base4 + NS], jnp.zeros((16,), jnp.int32),
                           mask=mask4)

        def bt_step(t, bcur):
            idx = t * BEAM + bcur
            plsc.store_scatter(rows_v, [base4 + (t - 1)], idx, mask=mask4)
            return plsc.load_gather(prev_v, [idx])

        def bt_body(k, bcur):
            t0 = NS - k * 8
            for u in range(8):
                bcur = bt_step(t0 - u, bcur)
            return bcur

        b_tail = lax.fori_loop(0, NS // 8, bt_body, b)
        for t in range(NS - (NS // 8) * 8, 0, -1):
            b_tail = bt_step(t, b_tail)

        rotm1 = jnp.bitwise_and(lane + 15, 16 - 1)
        for i in range(NBEST):
            def blk_body(v, carry, i=i):
                off = i * T + v * 16
                ivec = rows_v[pl.ds(off, 16)]
                tokbuf_v[pl.ds(off, 16)] = plsc.load_gather(tokens_v, [ivec])
                sc = plsc.load_gather(scores_v, [ivec])
                srot = sc.at[rotm1].get(mode="promise_in_bounds")
                prev_sc = jnp.where(lane == 0, carry, srot)
                tlsbuf_v[pl.ds(off, 16)] = sc - prev_sc
                return sc[15]

            lax.fori_loop(0, T // 16, blk_body, jnp.float32(0.0))

        pltpu.sync_copy(tokbuf_v, tok_hbm)
        pltpu.sync_copy(tlsbuf_v, tls_hbm)
        pltpu.sync_copy(rows_v, rows_hbm)


_stage1 = functools.partial(
    pl.kernel,
    out_type=[
        jax.ShapeDtypeStruct((16,), jnp.int32),
        jax.ShapeDtypeStruct((16,), jnp.float32),
        jax.ShapeDtypeStruct((ROWS,), jnp.int32),
        jax.ShapeDtypeStruct((ROWS,), jnp.float32),
        jax.ShapeDtypeStruct((ROWS,), jnp.int32),
    ],
    mesh=plsc.VectorSubcoreMesh(**_MESH),
    compiler_params=pltpu.CompilerParams(needs_layout_passes=False),
    scratch_types=[
        pltpu.VMEM((T * BEAM,), jnp.int32),
        pltpu.VMEM((T * BEAM,), jnp.float32),
        pltpu.VMEM((T * BEAM,), jnp.int32),
        pltpu.VMEM((ROWS,), jnp.int32),
        pltpu.VMEM((ROWS,), jnp.int32),
        pltpu.VMEM((ROWS,), jnp.float32),
        pltpu.VMEM((16,), jnp.int32),
        pltpu.VMEM((16,), jnp.float32),
    ],
)(_stage1_body)


_CHUNK = 16
_PER_W = ROWS // 32
_NCHUNK = _PER_W // _CHUNK


def _stage2_body(tw_hbm, rows_hbm, out_hbm, idx_v, buf0, buf1,
                 gsem0, gsem1, wsem0, wsem1):
    wid = lax.axis_index("s") * 2 + lax.axis_index("c")
    base = wid * _PER_W
    pltpu.sync_copy(rows_hbm.at[pl.ds(base, _PER_W)], idx_v)

    bufs = (buf0, buf1)
    gsems = (gsem0, gsem1)
    wsems = (wsem0, wsem1)

    def fire(c):
        return pltpu.async_copy(
            tw_hbm.at[idx_v.at[pl.ds(c * _CHUNK, _CHUNK)]],
            bufs[c & 1], gsems[c & 1])

    gcp = {0: fire(0)}
    wcp = {}
    for c in range(_NCHUNK):
        p = c & 1
        q = (c + 1) & 1
        if c + 1 < _NCHUNK:
            if c >= 1:
                wcp[q].wait()
            gcp[q] = fire(c + 1)
        gcp[p].wait()
        wcp[p] = pltpu.async_copy(
            bufs[p], out_hbm.at[pl.ds(base + c * _CHUNK, _CHUNK)], wsems[p])
    wcp[(_NCHUNK - 1) & 1].wait()


_stage2 = functools.partial(
    pl.kernel,
    out_type=jax.ShapeDtypeStruct((ROWS, SRC), jnp.float32),
    mesh=plsc.VectorSubcoreMesh(**_MESH),
    compiler_params=pltpu.CompilerParams(needs_layout_passes=False),
    scratch_types=[
        pltpu.VMEM((_PER_W,), jnp.int32),
        pltpu.VMEM((_CHUNK, SRC), jnp.float32),
        pltpu.VMEM((_CHUNK, SRC), jnp.float32),
        pltpu.SemaphoreType.DMA,
        pltpu.SemaphoreType.DMA,
        pltpu.SemaphoreType.DMA,
        pltpu.SemaphoreType.DMA,
    ],
)(_stage2_body)


def _tr_body(x0, x1, x2, x3, o0, o1, o2, o3):
    for x, o in ((x0, o0), (x1, o1), (x2, o2), (x3, o3)):
        o[...] = jnp.swapaxes(x[0], 0, 1)[:, :NS]


def _stage3(compact):
    in_specs = [
        pl.BlockSpec((1, T, 128), lambda s, k=k: (k, 0, s))
        for k in range(NBEST)
    ]
    out_specs = [
        pl.BlockSpec((128, NS), lambda s: (s, 0)) for _ in range(NBEST)
    ]
    out_shape = [
        jax.ShapeDtypeStruct((SRC, NS), jnp.float32) for _ in range(NBEST)
    ]
    return pl.pallas_call(
        _tr_body,
        grid=(SRC // 128,),
        in_specs=in_specs,
        out_specs=out_specs,
        out_shape=out_shape,
    )(compact, compact, compact, compact)


def kernel(beam_tokens, beam_scores, token_weights, beam_prev_indices,
           num_steps):
    tokens_flat = beam_tokens.reshape(-1)
    scores_flat = beam_scores.reshape(-1)
    prev_flat = beam_prev_indices.reshape(-1)
    tw_flat = token_weights.reshape(T * BEAM, SRC)

    ord16, sc16, tokf, tlsf, rows = _stage1(tokens_flat, scores_flat,
                                            prev_flat)
    compact = _stage2(tw_flat, rows)
    baw = _stage3(compact.reshape(NBEST, T, SRC))

    ns_t = jnp.asarray(num_steps, jnp.int32)
    ns_f = ns_t.astype(jnp.float32)
    tok4 = tokf.reshape(NBEST, T)
    tls4 = tlsf.reshape(NBEST, T)
    outs = []
    for i in range(NBEST):
        outs.extend([
            tok4[i, :NS],
            sc16[i] / ns_f,
            tls4[i, :NS],
            baw[i],
            jnp.stack([ns_t, ord16[i]]).astype(jnp.int32),
        ])
    return tuple(outs)
